# baseline (speedup 1.0000x reference)
import functools

import jax
import jax.numpy as jnp
from jax import lax
from jax.experimental import pallas as pl
from jax.experimental.pallas import tpu as pltpu
from jax.experimental.pallas import tpu_sc as plsc

SEQ = 200
EMBED_DIM = 64
VOCAB = 1000000
BATCH = 4096
LANES = 16
NUM_CORES = 2
NUM_SUBCORES = 16
NUM_WORKERS = NUM_CORES * NUM_SUBCORES
B_TOTAL = BATCH * SEQ
ROWS_PER_W = B_TOTAL // NUM_WORKERS
CHUNK = 128
NCH = ROWS_PER_W // CHUNK
NPAIR = NCH // 2
D_VECS = EMBED_DIM // LANES
SKEW = CHUNK + 1
SCALE = 8.0
ST = SEQ // 8
BT = BATCH // CHUNK


def _sc_body(idx_hbm, table_hbm, pos_hbm, out_hbm,
             idxa, idxa2, pos_v, rows0, rows1, outb0, outb1,
             semg0, semg1, semo0, semo1):
    wid = lax.axis_index("s") * NUM_CORES + lax.axis_index("c")
    base = wid * ROWS_PER_W
    pltpu.sync_copy(idx_hbm.at[pl.ds(base, ROWS_PER_W)], idxa)
    pltpu.sync_copy(pos_hbm, pos_v)
    iota = lax.iota(jnp.int32, LANES)
    jbase = wid * NCH

    @plsc.parallel_loop(0, ROWS_PER_W // LANES, unroll=8)
    def _shift(i):
        sl = pl.ds(i * LANES, LANES)
        idxa2[sl] = lax.shift_right_logical(idxa[sl], 1)

    def decomp(j):
        jj = jbase + j
        block = jj // 8
        s = (block // BT) * 8 + jj % 8
        bt = block % BT
        return s, bt

    def gather_start(j, rows, sem):
        pltpu.async_copy(table_hbm.at[idxa2.at[pl.ds(j * CHUNK, CHUNK)]],
                         rows, sem)

    def gather_wait(rows, sem):
        pltpu.make_async_copy(table_hbm.at[idxa2.at[pl.ds(0, CHUNK)]],
                              rows, sem).wait()

    def out_start(j, outb, sem):
        s, bt = decomp(j)
        pltpu.async_copy(outb.at[:, :, pl.ds(0, CHUNK)],
                         out_hbm.at[s, :, bt], sem)

    def out_wait(outb, sem):
        pltpu.make_async_copy(outb.at[:, :, pl.ds(0, CHUNK)],
                              out_hbm.at[0, :, 0], sem).wait()

    c1vecs = [(iota // 8) + 2 * k for k in range(D_VECS)]
    c2vec = iota % 8

    def compute(j, rows, outb):
        s, _ = decomp(j)
        pvecs = [pos_v[s, pl.ds(k * LANES, LANES)] for k in range(D_VECS)]

        @plsc.parallel_loop(0, CHUNK, unroll=4)
        def _r(r):
            rvec = jnp.full((LANES,), r, jnp.int32)
            idv = plsc.load_gather(idxa, [j * CHUNK + rvec])
            odd = (idv & 1) == 1
            for k in range(D_VECS):
                lo = rows[r, pl.ds(k * LANES, LANES)]
                hi = rows[r, pl.ds(EMBED_DIM + k * LANES, LANES)]
                v = jnp.where(odd, hi, lo)
                y = v * SCALE + pvecs[k]
                plsc.store_scatter(outb, [c1vecs[k], c2vec, rvec], y)

    gather_start(0, rows0, semg0)

    @pl.loop(0, NPAIR)
    def _pair(p):
        j0 = 2 * p
        gather_start(j0 + 1, rows1, semg1)
        gather_wait(rows0, semg0)

        @pl.when(p > 0)
        def _():
            out_wait(outb0, semo0)

        compute(j0, rows0, outb0)
        out_start(j0, outb0, semo0)

        @pl.when(p + 1 < NPAIR)
        def _():
            gather_start(j0 + 2, rows0, semg0)

        gather_wait(rows1, semg1)

        @pl.when(p > 0)
        def _():
            out_wait(outb1, semo1)

        compute(j0 + 1, rows1, outb1)
        out_start(j0 + 1, outb1, semo1)

    out_wait(outb0, semo0)
    out_wait(outb1, semo1)


@jax.jit
def _embed(idx_flat, table2, pos_table):
    grid_kernel = pl.kernel(
        _sc_body,
        out_type=jax.ShapeDtypeStruct((SEQ, EMBED_DIM // 8, BT, 8, CHUNK),
                                      jnp.float32),
        mesh=plsc.VectorSubcoreMesh(core_axis_name="c", subcore_axis_name="s"),
        scratch_types=[
            pltpu.VMEM((ROWS_PER_W,), jnp.int32),
            pltpu.VMEM((ROWS_PER_W,), jnp.int32),
            pltpu.VMEM((SEQ, EMBED_DIM), jnp.float32),
            pltpu.VMEM((CHUNK, 2 * EMBED_DIM), jnp.float32),
            pltpu.VMEM((CHUNK, 2 * EMBED_DIM), jnp.float32),
            pltpu.VMEM((EMBED_DIM // 8, 8, SKEW), jnp.float32),
            pltpu.VMEM((EMBED_DIM // 8, 8, SKEW), jnp.float32),
            pltpu.SemaphoreType.DMA,
            pltpu.SemaphoreType.DMA,
            pltpu.SemaphoreType.DMA,
            pltpu.SemaphoreType.DMA,
        ],
        compiler_params=pltpu.CompilerParams(
            use_tc_tiling_on_sc=False, needs_layout_passes=False),
    )
    return grid_kernel(idx_flat, table2, pos_table)


def kernel(inputs, token_table, pos_table):
    idx4 = inputs.astype(jnp.int32).reshape(BT, CHUNK, ST, 8)
    idx_flat = idx4.transpose(2, 0, 3, 1).reshape(-1)
    table2 = token_table.reshape(VOCAB // 2, 2 * EMBED_DIM)
    out5 = _embed(idx_flat, table2, pos_table)
    return out5.transpose(2, 4, 0, 1, 3).reshape(BATCH, SEQ, EMBED_DIM)


# speedup vs baseline: 1.1457x; 1.1457x over previous
import functools

import jax
import jax.numpy as jnp
from jax import lax
from jax.experimental import pallas as pl
from jax.experimental.pallas import tpu as pltpu
from jax.experimental.pallas import tpu_sc as plsc

SEQ = 200
EMBED_DIM = 64
BATCH = 4096
LANES = 16
NUM_CORES = 2
NUM_SUBCORES = 16
NUM_WORKERS = NUM_CORES * NUM_SUBCORES
B_TOTAL = BATCH * SEQ
ROWS_PER_W = B_TOTAL // NUM_WORKERS
CHUNK = 256
HALF = 128
NCH = ROWS_PER_W // CHUNK
NPAIR = NCH // 2
D_VECS = EMBED_DIM // LANES
SKEW = CHUNK + 1
SCALE = 8.0
ST = SEQ // 8
BT = BATCH // HALF


def _sc_body(idx_hbm, table_hbm, pos_hbm, out_hbm,
             idxa, pos_v, rows0, rows1, outb0, outb1,
             semg0, semg1, semo0, semo1):
    wid = lax.axis_index("s") * NUM_CORES + lax.axis_index("c")
    base = wid * ROWS_PER_W
    pltpu.sync_copy(idx_hbm.at[pl.ds(base, ROWS_PER_W)], idxa)
    pltpu.sync_copy(pos_hbm, pos_v)
    iota = lax.iota(jnp.int32, LANES)
    jbase = wid * NCH

    def decomp(j):
        jj = jbase + j
        block = jj // 4
        pair = jj % 4
        s0 = (block // BT) * 8 + pair * 2
        bt = block % BT
        return s0, bt

    def gather_start(j, rows, sem):
        pltpu.async_copy(table_hbm.at[idxa.at[pl.ds(j * CHUNK, CHUNK)]],
                         rows, sem)

    def gather_wait(rows, sem):
        pltpu.make_async_copy(table_hbm.at[idxa.at[pl.ds(0, CHUNK)]],
                              rows, sem).wait()

    def out_start(j, outb, sem):
        s0, bt = decomp(j)
        for h in range(2):
            pltpu.async_copy(outb.at[:, :, pl.ds(h * HALF, HALF)],
                             out_hbm.at[s0 + h, :, bt], sem)

    def out_wait(outb, sem):
        for h in range(2):
            pltpu.make_async_copy(outb.at[:, :, pl.ds(h * HALF, HALF)],
                                  out_hbm.at[0, :, 0], sem).wait()

    c1vecs = [(iota // 8) + 2 * k for k in range(D_VECS)]
    c2vec = iota % 8

    def compute(j, rows, outb):
        s0, _ = decomp(j)
        for h in range(2):
            s = s0 + h
            pvecs = [pos_v[s, pl.ds(k * LANES, LANES)] for k in range(D_VECS)]

            @plsc.parallel_loop(h * HALF, h * HALF + HALF, unroll=4)
            def _r(r):
                rvec = jnp.full((LANES,), r, jnp.int32)
                for k in range(D_VECS):
                    v = rows[r, pl.ds(k * LANES, LANES)]
                    y = v * SCALE + pvecs[k]
                    plsc.store_scatter(outb, [c1vecs[k], c2vec, rvec], y)

    gather_start(0, rows0, semg0)

    @pl.loop(0, NPAIR)
    def _pair(p):
        j0 = 2 * p
        gather_start(j0 + 1, rows1, semg1)
        gather_wait(rows0, semg0)

        @pl.when(p > 0)
        def _():
            out_wait(outb0, semo0)

        compute(j0, rows0, outb0)
        out_start(j0, outb0, semo0)

        @pl.when(p + 1 < NPAIR)
        def _():
            gather_start(j0 + 2, rows0, semg0)

        gather_wait(rows1, semg1)

        @pl.when(p > 0)
        def _():
            out_wait(outb1, semo1)

        compute(j0 + 1, rows1, outb1)
        out_start(j0 + 1, outb1, semo1)

    out_wait(outb0, semo0)
    out_wait(outb1, semo1)


@jax.jit
def _embed(idx_flat, token_table, pos_table):
    grid_kernel = pl.kernel(
        _sc_body,
        out_type=jax.ShapeDtypeStruct((SEQ, EMBED_DIM // 8, BT, 8, HALF),
                                      jnp.float32),
        mesh=plsc.VectorSubcoreMesh(core_axis_name="c", subcore_axis_name="s"),
        scratch_types=[
            pltpu.VMEM((ROWS_PER_W,), jnp.int32),
            pltpu.VMEM((SEQ, EMBED_DIM), jnp.float32),
            pltpu.VMEM((CHUNK, EMBED_DIM), jnp.float32),
            pltpu.VMEM((CHUNK, EMBED_DIM), jnp.float32),
            pltpu.VMEM((EMBED_DIM // 8, 8, SKEW), jnp.float32),
            pltpu.VMEM((EMBED_DIM // 8, 8, SKEW), jnp.float32),
            pltpu.SemaphoreType.DMA,
            pltpu.SemaphoreType.DMA,
            pltpu.SemaphoreType.DMA,
            pltpu.SemaphoreType.DMA,
        ],
        compiler_params=pltpu.CompilerParams(
            use_tc_tiling_on_sc=False, needs_layout_passes=False),
    )
    return grid_kernel(idx_flat, token_table, pos_table)


def kernel(inputs, token_table, pos_table):
    idx4 = inputs.astype(jnp.int32).reshape(BT, HALF, ST, 8)
    idx_flat = idx4.transpose(2, 0, 3, 1).reshape(-1)
    out5 = _embed(idx_flat, token_table, pos_table)
    return out5.transpose(2, 4, 0, 1, 3).reshape(BATCH, SEQ, EMBED_DIM)
